# baseline (speedup 1.0000x reference)
import functools

import jax
import jax.numpy as jnp
from jax import lax
from jax.experimental import pallas as pl
from jax.experimental.pallas import tpu as pltpu
from jax.experimental.pallas import tpu_sc as plsc

N_NODES = 10000
N_EDGES = 320000
D = 128

NC = 2
NS = 16
NW = NC * NS
CHUNK = 128
CPT = 80
HCPT = CPT // 2
EPT = CPT * CHUNK
E_PAD = NW * EPT
N_PAD = 10240
RPT = N_PAD // NS

_sc_mesh = plsc.VectorSubcoreMesh(core_axis_name="c", subcore_axis_name="s")


def _sc_agg_body(with_deg, table_h, src_h, dst_h, zeros_h, zeros1_h, out_h, deg_out_h,
                 acc_sh, deg_sh, sidx_all, didx_all, rows0, ones1, dz, gsem0):
    cid = lax.axis_index("c")
    sid = lax.axis_index("s")
    w = sid * NC + cid
    r0 = sid * RPT
    pltpu.sync_copy(zeros_h.at[pl.ds(0, CHUNK)], rows0)
    for k in range(RPT // CHUNK):
        pltpu.sync_copy(rows0, acc_sh.at[pl.ds(r0 + k * CHUNK, CHUNK)])
    pltpu.sync_copy(zeros1_h.at[pl.ds(r0, RPT)], dz)
    pltpu.sync_copy(dz, deg_sh.at[pl.ds(r0, RPT)])
    for j in range(CHUNK // 16):
        ones1[pl.ds(j * 16, 16)] = jnp.full((16,), 1.0, jnp.float32)
    plsc.subcore_barrier()

    for h in range(2):
        pltpu.sync_copy(src_h.at[pl.ds(w * CPT + h * HCPT, HCPT)], sidx_all)
        pltpu.sync_copy(dst_h.at[pl.ds(w * CPT + h * HCPT, HCPT)], didx_all)

        def chunk_body(c, carry):
            pltpu.async_copy(table_h.at[sidx_all.at[c]], rows0, gsem0).wait()
            pltpu.sync_copy(rows0, acc_sh.at[didx_all.at[c]], add=True)
            if with_deg:
                pltpu.sync_copy(ones1, deg_sh.at[didx_all.at[c]], add=True)
            return carry

        lax.fori_loop(0, HCPT, chunk_body, 0)
    plsc.subcore_barrier()
    for k in range(RPT // CHUNK):
        pltpu.sync_copy(acc_sh.at[pl.ds(r0 + k * CHUNK, CHUNK)], rows0)
        pltpu.sync_copy(rows0, out_h.at[cid, pl.ds(r0 + k * CHUNK, CHUNK)])
    pltpu.sync_copy(deg_sh.at[pl.ds(r0, RPT)], dz)
    pltpu.sync_copy(dz, deg_out_h.at[pl.ds(cid * N_PAD + r0, RPT)])


def _make_sc_agg(with_deg):
    return pl.kernel(
        functools.partial(_sc_agg_body, with_deg),
        mesh=_sc_mesh,
        out_type=[
            jax.ShapeDtypeStruct((NC, N_PAD, D), jnp.float32),
            jax.ShapeDtypeStruct((NC * N_PAD,), jnp.float32),
        ],
        scratch_types=[
            pltpu.VMEM_SHARED((N_PAD, D), jnp.float32),
            pltpu.VMEM_SHARED((N_PAD,), jnp.float32),
            pltpu.VMEM((HCPT, CHUNK), jnp.int32),
            pltpu.VMEM((HCPT, CHUNK), jnp.int32),
            pltpu.VMEM((CHUNK, D), jnp.float32),
            pltpu.VMEM((CHUNK,), jnp.float32),
            pltpu.VMEM((RPT,), jnp.float32),
            pltpu.SemaphoreType.DMA,
        ],
    )


_sc_agg_deg = _make_sc_agg(True)
_sc_agg_nodeg = _make_sc_agg(False)


def _tc_layer_body(p_ref, deg_ref, x_ref, wl_ref, bl_ref, wr_ref, o_ref):
    s = p_ref[0] + p_ref[1]
    agg = s / jnp.maximum(deg_ref[...], 1.0)
    h = (jnp.dot(agg, wl_ref[...], preferred_element_type=jnp.float32)
         + bl_ref[...]
         + jnp.dot(x_ref[...], wr_ref[...], preferred_element_type=jnp.float32))
    o_ref[...] = jnp.maximum(h, 0.0)


_TC_BLK = 1000


def _tc_layer(p, deg2d, x, W_l, b_l, W_r):
    grid = N_NODES // _TC_BLK
    return pl.pallas_call(
        _tc_layer_body,
        grid=(grid,),
        in_specs=[
            pl.BlockSpec((NC, _TC_BLK, D), lambda i: (0, i, 0)),
            pl.BlockSpec((_TC_BLK, 1), lambda i: (i, 0)),
            pl.BlockSpec((_TC_BLK, D), lambda i: (i, 0)),
            pl.BlockSpec((D, D), lambda i: (0, 0)),
            pl.BlockSpec((1, D), lambda i: (0, 0)),
            pl.BlockSpec((D, D), lambda i: (0, 0)),
        ],
        out_specs=pl.BlockSpec((_TC_BLK, D), lambda i: (i, 0)),
        out_shape=jax.ShapeDtypeStruct((N_NODES, D), jnp.float32),
    )(p, deg2d, x, W_l, b_l.reshape(1, D), W_r)


@jax.jit
def kernel(x, edge_index, W_l1, b_l1, W_r1, W_l2, b_l2, W_r2):
    ei = edge_index.astype(jnp.int32)
    pad = E_PAD - N_EDGES
    src1 = jnp.concatenate([ei[0], jnp.zeros((pad,), jnp.int32)]).reshape(NW * CPT, CHUNK)
    dst1 = jnp.concatenate([ei[1], jnp.full((pad,), N_NODES, jnp.int32)]).reshape(NW * CPT, CHUNK)
    zeros_rows = jnp.zeros((N_PAD, D), jnp.float32)
    zeros1 = jnp.zeros((N_PAD,), jnp.float32)

    p1, dflat = _sc_agg_deg(x, src1, dst1, zeros_rows, zeros1)
    d2 = dflat.reshape(NC, N_PAD)
    deg2d = (d2[0, :N_NODES] + d2[1, :N_NODES]).reshape(N_NODES, 1)
    h1 = _tc_layer(p1[:, :N_NODES], deg2d, x, W_l1, b_l1, W_r1)
    p2, _ = _sc_agg_nodeg(h1, src1, dst1, zeros_rows, zeros1)
    h2 = _tc_layer(p2[:, :N_NODES], deg2d, h1, W_l2, b_l2, W_r2)
    return h2


# speedup vs baseline: 1.4242x; 1.4242x over previous
import functools

import jax
import jax.numpy as jnp
from jax import lax
from jax.experimental import pallas as pl
from jax.experimental.pallas import tpu as pltpu
from jax.experimental.pallas import tpu_sc as plsc

N_NODES = 10000
N_EDGES = 320000
D = 128

NC = 2
NS = 16
NW = NC * NS
CHUNK = 128
CPT = 79
EPT = CPT * CHUNK
E_PAD = NW * EPT
N_PAD = 10240
RPT = N_PAD // NS

_sc_mesh = plsc.VectorSubcoreMesh(core_axis_name="c", subcore_axis_name="s")


def _sc_agg_body(table_h, src_h, dst_h, zeros_h, zeros1_h, out_h, deg_out_h,
                 acc_sh, deg_sh, sidx, didx, rows, ones1, dz, sem):
    cid = lax.axis_index("c")
    sid = lax.axis_index("s")
    w = sid * NC + cid
    r0 = sid * RPT
    pltpu.sync_copy(zeros_h.at[pl.ds(0, CHUNK)], rows)
    for k in range(RPT // CHUNK):
        pltpu.sync_copy(rows, acc_sh.at[pl.ds(r0 + k * CHUNK, CHUNK)])
    pltpu.sync_copy(zeros1_h.at[pl.ds(r0, RPT)], dz)
    pltpu.sync_copy(dz, deg_sh.at[pl.ds(r0, RPT)])
    for j in range(CHUNK // 16):
        ones1[pl.ds(j * 16, 16)] = jnp.full((16,), 1.0, jnp.float32)
    plsc.subcore_barrier()

    e0 = w * EPT

    def chunk_body(c, carry):
        base = e0 + c * CHUNK
        pltpu.sync_copy(src_h.at[pl.ds(base, CHUNK)], sidx)
        pltpu.sync_copy(dst_h.at[pl.ds(base, CHUNK)], didx)
        pltpu.async_copy(table_h.at[sidx], rows, sem).wait()
        pltpu.sync_copy(rows, acc_sh.at[didx], add=True)
        pltpu.sync_copy(ones1, deg_sh.at[didx], add=True)
        return carry

    lax.fori_loop(0, CPT, chunk_body, 0)
    plsc.subcore_barrier()
    for k in range(RPT // CHUNK):
        pltpu.sync_copy(acc_sh.at[pl.ds(r0 + k * CHUNK, CHUNK)], rows)
        pltpu.sync_copy(rows, out_h.at[cid, pl.ds(r0 + k * CHUNK, CHUNK)])
    pltpu.sync_copy(deg_sh.at[pl.ds(r0, RPT)], dz)
    pltpu.sync_copy(dz, deg_out_h.at[pl.ds(cid * N_PAD + r0, RPT)])


_sc_agg = pl.kernel(
    _sc_agg_body,
    mesh=_sc_mesh,
    out_type=[
        jax.ShapeDtypeStruct((NC, N_PAD, D), jnp.float32),
        jax.ShapeDtypeStruct((NC * N_PAD,), jnp.float32),
    ],
    scratch_types=[
        pltpu.VMEM_SHARED((N_PAD, D), jnp.float32),
        pltpu.VMEM_SHARED((N_PAD,), jnp.float32),
        pltpu.VMEM((CHUNK,), jnp.int32),
        pltpu.VMEM((CHUNK,), jnp.int32),
        pltpu.VMEM((CHUNK, D), jnp.float32),
        pltpu.VMEM((CHUNK,), jnp.float32),
        pltpu.VMEM((RPT,), jnp.float32),
        pltpu.SemaphoreType.DMA,
    ],
)


def _tc_layer_body(p_ref, deg_ref, x_ref, wl_ref, bl_ref, wr_ref, o_ref):
    s = p_ref[0] + p_ref[1]
    agg = s / jnp.maximum(deg_ref[...], 1.0)
    h = (jnp.dot(agg, wl_ref[...], preferred_element_type=jnp.float32)
         + bl_ref[...]
         + jnp.dot(x_ref[...], wr_ref[...], preferred_element_type=jnp.float32))
    o_ref[...] = jnp.maximum(h, 0.0)


_TC_BLK = 1000


def _tc_layer(p, deg2d, x, W_l, b_l, W_r):
    grid = N_NODES // _TC_BLK
    return pl.pallas_call(
        _tc_layer_body,
        grid=(grid,),
        in_specs=[
            pl.BlockSpec((NC, _TC_BLK, D), lambda i: (0, i, 0)),
            pl.BlockSpec((_TC_BLK, 1), lambda i: (i, 0)),
            pl.BlockSpec((_TC_BLK, D), lambda i: (i, 0)),
            pl.BlockSpec((D, D), lambda i: (0, 0)),
            pl.BlockSpec((1, D), lambda i: (0, 0)),
            pl.BlockSpec((D, D), lambda i: (0, 0)),
        ],
        out_specs=pl.BlockSpec((_TC_BLK, D), lambda i: (i, 0)),
        out_shape=jax.ShapeDtypeStruct((N_NODES, D), jnp.float32),
    )(p, deg2d, x, W_l, b_l.reshape(1, D), W_r)


@jax.jit
def kernel(x, edge_index, W_l1, b_l1, W_r1, W_l2, b_l2, W_r2):
    ei = edge_index.astype(jnp.int32)
    pad = E_PAD - N_EDGES
    src1 = jnp.concatenate([ei[0], jnp.zeros((pad,), jnp.int32)])
    dst1 = jnp.concatenate([ei[1], jnp.full((pad,), N_NODES, jnp.int32)])
    zeros_rows = jnp.zeros((N_PAD, D), jnp.float32)
    zeros1 = jnp.zeros((N_PAD,), jnp.float32)

    p1, dflat = _sc_agg(x, src1, dst1, zeros_rows, zeros1)
    d2 = dflat.reshape(NC, N_PAD)
    deg2d = (d2[0, :N_NODES] + d2[1, :N_NODES]).reshape(N_NODES, 1)
    h1 = _tc_layer(p1[:, :N_NODES], deg2d, x, W_l1, b_l1, W_r1)
    p2, _ = _sc_agg(h1, src1, dst1, zeros_rows, zeros1)
    h2 = _tc_layer(p2[:, :N_NODES], deg2d, h1, W_l2, b_l2, W_r2)
    return h2
